# Initial kernel scaffold; baseline (speedup 1.0000x reference)
import jax
import jax.numpy as jnp
from jax.experimental import pallas as pl


def kernel(x_nchw, w1s, b1s, w2s, b2s, G, wf1, bf1, wf2, bf2, wf3, bf3):
    raise NotImplementedError("write your pallas kernel here")


# speedup vs baseline: 7.8001x; 7.8001x over previous
import numpy as np

import jax
import jax.numpy as jnp
from jax.experimental import pallas as pl
from jax.experimental.pallas import tpu as pltpu


def _gather_1024_to_256():
    g = np.zeros((1024, 256), np.float32)
    for p in range(15):
        for q in range(15):
            g[64 * p + 2 * q, 16 * p + q] = 1.0
    return g


_G1_NP = _gather_1024_to_256()


def _fused_kernel(x_ref, w1_ref, b1_ref, w2_ref, b2_ref, g1_ref,
                  wf1_ref, bf1_ref, wf2_ref, bf2_ref, wf3_ref, bf3_ref,
                  out_ref, t1_ref, t2_ref):
    f32 = jnp.float32

    def shl2(a, s):
        if s == 0:
            return a
        return jnp.concatenate([a[:, s:], a[:, :s]], axis=1)

    def shl3(a, s):
        if s == 0:
            return a
        return jnp.concatenate([a[..., s:], a[..., :s]], axis=2)

    x = x_ref[...]
    for t in range(9):
        dh, dw = divmod(t, 3)
        t1_ref[t] = shl2(x, 32 * dh + dw)

    r1 = jnp.einsum('ck,kbn->cbn', w1_ref[...], t1_ref[...],
                    preferred_element_type=f32)
    r1 = r1 + b1_ref[...]
    m1 = jnp.maximum(r1, shl3(r1, 1))
    y1 = jnp.maximum(jnp.maximum(m1, shl3(m1, 32)), 0.0)

    d3 = jax.lax.dot_general(y1, g1_ref[...], (((2,), (0,)), ((), ())),
                             preferred_element_type=f32)

    for ci in range(6):
        dci = d3[ci]
        for t in range(9):
            dh, dw = divmod(t, 3)
            t2_ref[ci * 9 + t] = shl2(dci, 16 * dh + dw)

    r2 = jnp.einsum('ok,kbn->obn', w2_ref[...], t2_ref[...],
                    preferred_element_type=f32)
    r2 = r2 + b2_ref[...]
    m2 = jnp.maximum(r2, shl3(r2, 1))
    y2 = jnp.maximum(jnp.maximum(m2, shl3(m2, 16)), 0.0)

    h = None
    for co in range(16):
        part = jnp.dot(y2[co], wf1_ref[co], preferred_element_type=f32)
        h = part if h is None else h + part
    h = jnp.maximum(h + bf1_ref[...], 0.0)
    h = jnp.maximum(jnp.dot(h, wf2_ref[...], preferred_element_type=f32)
                    + bf2_ref[...], 0.0)
    out_ref[...] = (jnp.dot(h, wf3_ref[...], preferred_element_type=f32)
                    + bf3_ref[...])


def kernel(x_nchw, w1s, b1s, w2s, b2s, G, wf1, bf1, wf2, bf2, wf3, bf3):
    del G
    b, c, hh, ww = x_nchw.shape
    assert (c, hh, ww) == (1, 32, 32)
    bt = 128

    x2d = x_nchw.reshape(b, 1024).astype(jnp.float32)
    b_pad = ((b + bt - 1) // bt) * bt
    if b_pad != b:
        x2d = jnp.pad(x2d, ((0, b_pad - b), (0, 0)))

    f32 = jnp.float32
    b1v = jnp.broadcast_to(b1s.astype(f32).reshape(6, 1, 1), (6, 1, 1024))
    b2v = jnp.broadcast_to(b2s.astype(f32).reshape(16, 1, 1), (16, 1, 256))
    w4 = wf1.astype(f32).reshape(16, 8, 8, 120)
    wf1d = jnp.zeros((16, 16, 16, 120), f32)
    wf1d = wf1d.at[:, 0:12:2, 0:12:2, :].set(w4[:, :6, :6, :])
    wf1d = wf1d.reshape(16, 256, 120)
    g1 = jnp.asarray(_G1_NP)

    vmem_ws = [w1s.astype(f32), b1v, w2s.astype(f32), b2v, g1,
               wf1d, bf1.astype(f32), wf2.astype(f32), bf2.astype(f32),
               wf3.astype(f32), bf3.astype(f32)]

    def resident(a):
        n = a.ndim
        return pl.BlockSpec(a.shape, lambda i, _n=n: (0,) * _n)

    out = pl.pallas_call(
        _fused_kernel,
        grid=(b_pad // bt,),
        in_specs=([pl.BlockSpec((bt, 1024), lambda i: (i, 0))]
                  + [resident(a) for a in vmem_ws]),
        out_specs=pl.BlockSpec((bt, 128), lambda i: (i, 0)),
        out_shape=jax.ShapeDtypeStruct((b_pad, 128), f32),
        scratch_shapes=[pltpu.VMEM((9, bt, 1024), f32),
                        pltpu.VMEM((54, bt, 256), f32)],
        compiler_params=pltpu.CompilerParams(
            dimension_semantics=("parallel",),
            vmem_limit_bytes=48 * 1024 * 1024),
    )(x2d, *vmem_ws)
    return out[:b, :10]


# speedup vs baseline: 11.2081x; 1.4369x over previous
import numpy as np

import jax
import jax.numpy as jnp
from jax.experimental import pallas as pl
from jax.experimental.pallas import tpu as pltpu


def _gather_1024_to_256():
    g = np.zeros((1024, 256), np.float32)
    for p in range(15):
        for q in range(15):
            g[64 * p + 2 * q, 16 * p + q] = 1.0
    return g


_G1_NP = _gather_1024_to_256()


def _fused_kernel(x_ref, w1_ref, b1_ref, w2_ref, b2_ref, g1_ref,
                  wf1_ref, bf1_ref, wf2_ref, bf2_ref, wf3_ref, bf3_ref,
                  out_ref, t2_ref):
    f32 = jnp.float32

    def shl2(a, s):
        if s == 0:
            return a
        return jnp.concatenate([a[:, s:], a[:, :s]], axis=1)

    def shl3(a, s):
        if s == 0:
            return a
        return jnp.concatenate([a[..., s:], a[..., :s]], axis=2)

    bf16 = jnp.bfloat16
    x = x_ref[...]
    taps = [shl2(x, 32 * dh + dw) for dh in range(3) for dw in range(3)]
    y1 = []
    for c in range(6):
        r1 = b1_ref[c] + w1_ref[c, 0] * taps[0]
        for t in range(1, 9):
            r1 = r1 + w1_ref[c, t] * taps[t]
        m1 = jnp.maximum(r1, shl2(r1, 1))
        y1.append(jnp.maximum(jnp.maximum(m1, shl2(m1, 32)), 0.0)
                  .astype(bf16))

    d3 = [jnp.dot(y1c, g1_ref[...], preferred_element_type=f32)
          for y1c in y1]

    for ci in range(6):
        dci = d3[ci]
        for t in range(9):
            dh, dw = divmod(t, 3)
            t2_ref[ci * 9 + t] = shl2(dci, 16 * dh + dw).astype(bf16)

    r2 = jnp.einsum('ok,kbn->obn', w2_ref[...], t2_ref[...],
                    preferred_element_type=f32)
    r2 = r2 + b2_ref[...]
    m2 = jnp.maximum(r2, shl3(r2, 1))
    y2 = jnp.maximum(jnp.maximum(m2, shl3(m2, 16)), 0.0)

    h = None
    for co in range(16):
        part = jnp.dot(y2[co].astype(bf16), wf1_ref[co],
                       preferred_element_type=f32)
        h = part if h is None else h + part
    h = jnp.maximum(h + bf1_ref[...], 0.0)
    h = jnp.maximum(jnp.dot(h, wf2_ref[...], preferred_element_type=f32)
                    + bf2_ref[...], 0.0)
    out_ref[...] = (jnp.dot(h, wf3_ref[...], preferred_element_type=f32)
                    + bf3_ref[...])


def kernel(x_nchw, w1s, b1s, w2s, b2s, G, wf1, bf1, wf2, bf2, wf3, bf3):
    del G
    b, c, hh, ww = x_nchw.shape
    assert (c, hh, ww) == (1, 32, 32)
    bt = 128

    x2d = x_nchw.reshape(b, 1024).astype(jnp.float32)
    b_pad = ((b + bt - 1) // bt) * bt
    if b_pad != b:
        x2d = jnp.pad(x2d, ((0, b_pad - b), (0, 0)))

    f32 = jnp.float32
    b2v = jnp.broadcast_to(b2s.astype(f32).reshape(16, 1, 1), (16, 1, 256))
    w4 = wf1.astype(f32).reshape(16, 8, 8, 120)
    wf1d = jnp.zeros((16, 16, 16, 120), f32)
    wf1d = wf1d.at[:, 0:12:2, 0:12:2, :].set(w4[:, :6, :6, :])
    wf1d = wf1d.reshape(16, 256, 120)
    g1 = jnp.asarray(_G1_NP)

    bf = jnp.bfloat16
    smem_ws = [w1s.astype(f32), b1s.astype(f32)]
    vmem_ws = [w2s.astype(bf), b2v, g1.astype(bf),
               wf1d.astype(bf), bf1.astype(f32), wf2.astype(f32),
               bf2.astype(f32), wf3.astype(f32), bf3.astype(f32)]

    def resident(a):
        n = a.ndim
        return pl.BlockSpec(a.shape, lambda i, _n=n: (0,) * _n)

    smem_spec = pl.BlockSpec(memory_space=pltpu.MemorySpace.SMEM)
    out = pl.pallas_call(
        _fused_kernel,
        grid=(b_pad // bt,),
        in_specs=([pl.BlockSpec((bt, 1024), lambda i: (i, 0))]
                  + [smem_spec] * len(smem_ws)
                  + [resident(a) for a in vmem_ws]),
        out_specs=pl.BlockSpec((bt, 128), lambda i: (i, 0)),
        out_shape=jax.ShapeDtypeStruct((b_pad, 128), f32),
        scratch_shapes=[pltpu.VMEM((54, bt, 256), jnp.bfloat16)],
        compiler_params=pltpu.CompilerParams(
            dimension_semantics=("parallel",),
            vmem_limit_bytes=48 * 1024 * 1024),
    )(x2d, *smem_ws, *vmem_ws)
    return out[:b, :10]


# speedup vs baseline: 13.5721x; 1.2109x over previous
import numpy as np

import jax
import jax.numpy as jnp
from jax.experimental import pallas as pl
from jax.experimental.pallas import tpu as pltpu


def _conv1_basis():
    b = np.zeros((9, 1024, 256), np.float32)
    for dh in range(3):
        for dw in range(3):
            for p in range(15):
                for q in range(15):
                    b[dh * 3 + dw, 64 * p + 2 * q + 32 * dh + dw,
                      16 * p + q] = 1.0
    return b


def _conv2_basis():
    b = np.zeros((9, 256, 64), np.float32)
    for dh in range(3):
        for dw in range(3):
            for u in range(6):
                for v in range(6):
                    b[dh * 3 + dw, 16 * (2 * u + dh) + (2 * v + dw),
                      8 * u + v] = 1.0
    return b


_B1_NP = _conv1_basis()
_B2_NP = _conv2_basis()


def _fused_kernel(x_ref, m4_ref, b1_ref, m2p_ref, b2_ref,
                  wf1_ref, bf1_ref, wf2_ref, bf2_ref, wf3_ref, bf3_ref,
                  out_ref):
    f32 = jnp.float32
    bf16 = jnp.bfloat16

    p4 = jnp.dot(x_ref[...].astype(bf16), m4_ref[...],
                 preferred_element_type=f32)
    m1 = jnp.maximum(jnp.maximum(p4[:, 0:1536], p4[:, 1536:3072]),
                     jnp.maximum(p4[:, 3072:4608], p4[:, 4608:6144]))
    d = jnp.maximum(m1 + b1_ref[...], 0.0)

    p2 = jnp.dot(d.astype(bf16), m2p_ref[...],
                 preferred_element_type=f32)
    m2 = jnp.maximum(jnp.maximum(p2[:, 0:1024], p2[:, 1024:2048]),
                     jnp.maximum(p2[:, 2048:3072], p2[:, 3072:4096]))
    feats = jnp.maximum(m2 + b2_ref[...], 0.0)

    h = jnp.maximum(jnp.dot(feats.astype(bf16), wf1_ref[...],
                            preferred_element_type=f32)
                    + bf1_ref[...], 0.0)
    h = jnp.maximum(jnp.dot(h, wf2_ref[...], preferred_element_type=f32)
                    + bf2_ref[...], 0.0)
    out_ref[...] = (jnp.dot(h, wf3_ref[...], preferred_element_type=f32)
                    + bf3_ref[...])


def kernel(x_nchw, w1s, b1s, w2s, b2s, G, wf1, bf1, wf2, bf2, wf3, bf3):
    del G
    b, c, hh, ww = x_nchw.shape
    assert (c, hh, ww) == (1, 32, 32)
    bt = 256

    f32 = jnp.float32
    bf = jnp.bfloat16

    x2d = x_nchw.reshape(b, 1024).astype(f32)
    b_pad = ((b + bt - 1) // bt) * bt
    if b_pad != b:
        x2d = jnp.pad(x2d, ((0, b_pad - b), (0, 0)))

    m0 = jnp.einsum('ct,tmn->mcn', w1s.astype(bf), jnp.asarray(_B1_NP, bf),
                    preferred_element_type=f32)
    m0 = m0.reshape(1024, 1536).astype(bf)
    m4 = jnp.stack([jnp.roll(m0, off, axis=0) for off in (0, 1, 32, 33)],
                   axis=1).reshape(1024, 4 * 1536)

    w2r = w2s.astype(bf).reshape(16, 6, 9)
    m20 = jnp.einsum('oct,tms->cmos', w2r, jnp.asarray(_B2_NP, bf),
                     preferred_element_type=f32)
    m20 = m20.reshape(1536, 1024).astype(bf)
    m2p = jnp.stack([jnp.roll(m20, 16 * kr + kc, axis=0)
                     for kr in (0, 1) for kc in (0, 1)],
                    axis=1).reshape(1536, 4 * 1024)

    b1cat = jnp.repeat(b1s.astype(f32), 256).reshape(1, 1536)
    b2cat = jnp.repeat(b2s.astype(f32), 64).reshape(1, 1024)

    vmem_ws = [m4, b1cat, m2p, b2cat,
               wf1.astype(bf), bf1.astype(f32), wf2.astype(f32),
               bf2.astype(f32), wf3.astype(f32), bf3.astype(f32)]

    def resident(a):
        n = a.ndim
        return pl.BlockSpec(a.shape, lambda i, _n=n: (0,) * _n)

    out = pl.pallas_call(
        _fused_kernel,
        grid=(b_pad // bt,),
        in_specs=([pl.BlockSpec((bt, 1024), lambda i: (i, 0))]
                  + [resident(a) for a in vmem_ws]),
        out_specs=pl.BlockSpec((bt, 128), lambda i: (i, 0)),
        out_shape=jax.ShapeDtypeStruct((b_pad, 128), f32),
        compiler_params=pltpu.CompilerParams(
            dimension_semantics=("parallel",),
            vmem_limit_bytes=56 * 1024 * 1024),
    )(x2d, *vmem_ws)
    return out[:b, :10]
